# baseline (speedup 1.0000x reference)
import jax
import jax.numpy as jnp
from jax.experimental import pallas as pl

_N = 400000
_M = 40000
_VOX = 0.1
_EPS = 1e-5
_T = 2000


def _stats_k(f_ref, s_ref, g_ref):
    @pl.when(pl.program_id(0) == 0)
    def _():
        s_ref[...] = jnp.zeros_like(s_ref)
        g_ref[...] = jnp.zeros_like(g_ref)

    x = f_ref[...]
    s_ref[...] += jnp.sum(x, axis=0, keepdims=True)
    g_ref[...] += jnp.dot(x.T, x, preferred_element_type=jnp.float32)


def _l1_k(f_ref, w_ref, d_ref, o_ref, s_ref, g_ref):
    @pl.when(pl.program_id(0) == 0)
    def _():
        s_ref[...] = jnp.zeros_like(s_ref)
        g_ref[...] = jnp.zeros_like(g_ref)

    x = jnp.dot(f_ref[...], w_ref[...], preferred_element_type=jnp.float32)
    x = jnp.maximum(x + d_ref[...], 0.0)
    o_ref[...] = x
    s_ref[...] += jnp.sum(x, axis=0, keepdims=True)
    g_ref[...] += jnp.dot(x.T, x, preferred_element_type=jnp.float32)


def _l23_k(x_ref, wa_ref, da_ref, w3_ref, b3_ref, o_ref):
    t = jnp.dot(x_ref[...], wa_ref[...], preferred_element_type=jnp.float32)
    t = jnp.maximum(t + da_ref[...], 0.0)
    o_ref[...] = jnp.dot(t, w3_ref[...], preferred_element_type=jnp.float32) + b3_ref[...]


def kernel(points, full_coors, coors_inv, normal, bn0_g, bn0_b, W1, b1,
           bn1_g, bn1_b, W2, b2, bn2_g, bn2_b, W3, b3):
    nt = _N // _T
    f32 = jnp.float32

    pts3 = points[:, :3]
    ones = jnp.ones((_N, 1), dtype=f32)
    cnt = jax.ops.segment_sum(ones, coors_inv, num_segments=_M)
    cnt = jnp.maximum(cnt, 1.0)
    seg_sum = jax.ops.segment_sum(pts3, coors_inv, num_segments=_M)
    pc_mean = (seg_sum / cnt)[coors_inv]
    nor_pc = pts3 - pc_mean
    min_volume_space = jnp.floor(jnp.min(pts3, axis=0))
    voxel_centers = full_coors[:, 1:].astype(f32) * _VOX + min_volume_space
    center_to_point = pts3 - voxel_centers
    f = jnp.concatenate([points, nor_pc, center_to_point, normal], axis=1)

    s0, g0 = pl.pallas_call(
        _stats_k,
        grid=(nt,),
        in_specs=[pl.BlockSpec((_T, 13), lambda i: (i, 0))],
        out_specs=[pl.BlockSpec((1, 13), lambda i: (0, 0)),
                   pl.BlockSpec((13, 13), lambda i: (0, 0))],
        out_shape=[jax.ShapeDtypeStruct((1, 13), f32),
                   jax.ShapeDtypeStruct((13, 13), f32)],
    )(f)
    mu_f = s0[0] / _N
    cov_f = g0 / _N - mu_f[:, None] * mu_f[None, :]
    var_f = jnp.diagonal(cov_f)

    a0 = bn0_g / jnp.sqrt(var_f + _EPS)
    c0 = bn0_b - mu_f * a0
    W1p = a0[:, None] * W1
    b1p = c0 @ W1 + b1
    mean1 = mu_f @ W1p + b1p
    var1 = jnp.sum(W1p * (cov_f @ W1p), axis=0)
    a1 = bn1_g / jnp.sqrt(var1 + _EPS)
    c1 = bn1_b - mean1 * a1
    Wf = W1p * a1[None, :]
    d1 = (b1p * a1 + c1)[None, :]

    x1n, s1, g1 = pl.pallas_call(
        _l1_k,
        grid=(nt,),
        in_specs=[pl.BlockSpec((_T, 13), lambda i: (i, 0)),
                  pl.BlockSpec((13, 64), lambda i: (0, 0)),
                  pl.BlockSpec((1, 64), lambda i: (0, 0))],
        out_specs=[pl.BlockSpec((_T, 64), lambda i: (i, 0)),
                   pl.BlockSpec((1, 64), lambda i: (0, 0)),
                   pl.BlockSpec((64, 64), lambda i: (0, 0))],
        out_shape=[jax.ShapeDtypeStruct((_N, 64), f32),
                   jax.ShapeDtypeStruct((1, 64), f32),
                   jax.ShapeDtypeStruct((64, 64), f32)],
    )(f, Wf, d1)
    m1 = s1[0] / _N
    cov1 = g1 / _N - m1[:, None] * m1[None, :]
    mean2 = m1 @ W2 + b2
    var2 = jnp.sum(W2 * (cov1 @ W2), axis=0)
    a2 = bn2_g / jnp.sqrt(var2 + _EPS)
    c2 = bn2_b - mean2 * a2
    Wa = W2 * a2[None, :]
    da = (b2 * a2 + c2)[None, :]

    x3 = pl.pallas_call(
        _l23_k,
        grid=(nt,),
        in_specs=[pl.BlockSpec((_T, 64), lambda i: (i, 0)),
                  pl.BlockSpec((64, 64), lambda i: (0, 0)),
                  pl.BlockSpec((1, 64), lambda i: (0, 0)),
                  pl.BlockSpec((64, 64), lambda i: (0, 0)),
                  pl.BlockSpec((1, 64), lambda i: (0, 0))],
        out_specs=pl.BlockSpec((_T, 64), lambda i: (i, 0)),
        out_shape=jax.ShapeDtypeStruct((_N, 64), f32),
    )(x1n, Wa, da, W3, b3[None, :])
    feat_sum = jax.ops.segment_sum(x3, coors_inv, num_segments=_M)
    return feat_sum / cnt
